# Initial kernel scaffold; baseline (speedup 1.0000x reference)
import jax
import jax.numpy as jnp
from jax.experimental import pallas as pl


def kernel(x, Wk, bk, Wq, bq, Wv, bv, Wu, bu):
    raise NotImplementedError("write your pallas kernel here")


# speedup vs baseline: 257.2847x; 257.2847x over previous
import jax
import jax.numpy as jnp
from jax import lax
from jax.experimental import pallas as pl
from jax.experimental.pallas import tpu as pltpu

_EMB = 768
_T = 2048
_KK = 32
_BLK = 256
_NB = _T // _BLK
_NM = _T // _KK


def _attn_kernel(x_ref, xs_ref, WkT_ref, WqT_ref, WvT_ref, Wu1T_ref, Wu2T_ref,
                 bk_ref, bq_ref, bv_ref, bu_ref, out_ref, qs_ref, vs_ref):
    i = pl.program_id(1)

    @pl.when(i == 0)
    def _():
        xs = xs_ref[0]
        qs_ref[...] = jnp.dot(xs, WqT_ref[...],
                              preferred_element_type=jnp.float32) + bq_ref[...]
        vs_ref[...] = jnp.dot(xs, WvT_ref[...],
                              preferred_element_type=jnp.float32) + bv_ref[...]

    xb = x_ref[0]
    K = jnp.dot(xb, WkT_ref[...], preferred_element_type=jnp.float32) + bk_ref[...]
    Q = jnp.dot(xb, WqT_ref[...], preferred_element_type=jnp.float32) + bq_ref[...]
    V = jnp.dot(xb, WvT_ref[...], preferred_element_type=jnp.float32) + bv_ref[...]

    S = jnp.dot(K, Q.T, preferred_element_type=jnp.float32)
    r = lax.broadcasted_iota(jnp.int32, (_BLK, _BLK), 0)
    c = lax.broadcasted_iota(jnp.int32, (_BLK, _BLK), 1)
    mask1 = (r // _KK == c // _KK) & (c <= r)
    h1 = jnp.dot(jnp.where(mask1, S, 0.0), V, preferred_element_type=jnp.float32)

    D = jnp.dot(K, qs_ref[...].T, preferred_element_type=jnp.float32)
    rj = lax.broadcasted_iota(jnp.int32, (_BLK, _NM), 0)
    cm = lax.broadcasted_iota(jnp.int32, (_BLK, _NM), 1)
    mask2 = (cm * _KK) <= (i * _BLK + rj)
    h2 = jnp.dot(jnp.where(mask2, D, 0.0), vs_ref[...],
                 preferred_element_type=jnp.float32)

    out_ref[0] = (jnp.dot(h1, Wu1T_ref[...], preferred_element_type=jnp.float32)
                  + jnp.dot(h2, Wu2T_ref[...], preferred_element_type=jnp.float32)
                  + bu_ref[...])


def kernel(x, Wk, bk, Wq, bq, Wv, bv, Wu, bu):
    B = x.shape[0]
    xs = x[:, ::_KK, :]
    wspec = pl.BlockSpec((_EMB, _EMB), lambda b, i: (0, 0))
    bspec = pl.BlockSpec((1, _EMB), lambda b, i: (0, 0))
    return pl.pallas_call(
        _attn_kernel,
        grid=(B, _NB),
        in_specs=[
            pl.BlockSpec((1, _BLK, _EMB), lambda b, i: (b, i, 0)),
            pl.BlockSpec((1, _NM, _EMB), lambda b, i: (b, 0, 0)),
            wspec, wspec, wspec, wspec, wspec,
            bspec, bspec, bspec, bspec,
        ],
        out_specs=pl.BlockSpec((1, _BLK, _EMB), lambda b, i: (b, i, 0)),
        out_shape=jax.ShapeDtypeStruct((B, _T, _EMB), jnp.float32),
        scratch_shapes=[
            pltpu.VMEM((_NM, _EMB), jnp.float32),
            pltpu.VMEM((_NM, _EMB), jnp.float32),
        ],
    )(x, xs, Wk.T, Wq.T, Wv.T, Wu[:, :_EMB].T, Wu[:, _EMB:].T,
      bk.reshape(1, _EMB), bq.reshape(1, _EMB), bv.reshape(1, _EMB),
      bu.reshape(1, _EMB))


# speedup vs baseline: 269.1398x; 1.0461x over previous
import jax
import jax.numpy as jnp
from jax import lax
from jax.experimental import pallas as pl
from jax.experimental.pallas import tpu as pltpu

_EMB = 768
_T = 2048
_KK = 32
_BLK = 256
_NB = _T // _BLK
_MPB = _BLK // _KK
_NM = _T // _KK
_BF = jnp.bfloat16


def _dot(a, b):
    return jnp.dot(a, b, preferred_element_type=jnp.float32)


def _attn_kernel(x_ref, WkT_ref, WqT_ref, WvT_ref, Wu1T_ref, Wu2T_ref,
                 bk_ref, bq_ref, bv_ref, bu_ref, out_ref, qs_ref, vs_ref):
    i = pl.program_id(1)

    @pl.when(i == 0)
    def _():
        qs_ref[...] = jnp.zeros((_NM, _EMB), _BF)
        vs_ref[...] = jnp.zeros((_NM, _EMB), _BF)

    xb = x_ref[0]
    K = _dot(xb, WkT_ref[...]) + bk_ref[...]
    Q = _dot(xb, WqT_ref[...]) + bq_ref[...]
    V = _dot(xb, WvT_ref[...]) + bv_ref[...]
    Kb = K.astype(_BF)
    Qb = Q.astype(_BF)
    Vb = V.astype(_BF)

    qs_ref[pl.ds(i * _MPB, _MPB), :] = Qb.reshape(_MPB, _KK, _EMB)[:, 0, :]
    vs_ref[pl.ds(i * _MPB, _MPB), :] = Vb.reshape(_MPB, _KK, _EMB)[:, 0, :]

    S = _dot(Kb, Qb.T)
    r = lax.broadcasted_iota(jnp.int32, (_BLK, _BLK), 0)
    c = lax.broadcasted_iota(jnp.int32, (_BLK, _BLK), 1)
    mask1 = (r // _KK == c // _KK) & (c <= r)
    h1 = _dot(jnp.where(mask1, S, 0.0).astype(_BF), Vb)

    D = _dot(Kb, qs_ref[...].T)
    rj = lax.broadcasted_iota(jnp.int32, (_BLK, _NM), 0)
    cm = lax.broadcasted_iota(jnp.int32, (_BLK, _NM), 1)
    mask2 = (cm * _KK) <= (i * _BLK + rj)
    h2 = _dot(jnp.where(mask2, D, 0.0).astype(_BF), vs_ref[...])

    out_ref[0] = (_dot(h1.astype(_BF), Wu1T_ref[...])
                  + _dot(h2.astype(_BF), Wu2T_ref[...])
                  + bu_ref[...])


def kernel(x, Wk, bk, Wq, bq, Wv, bv, Wu, bu):
    B = x.shape[0]
    wspec = pl.BlockSpec((_EMB, _EMB), lambda b, i: (0, 0))
    bspec = pl.BlockSpec((1, _EMB), lambda b, i: (0, 0))
    return pl.pallas_call(
        _attn_kernel,
        grid=(B, _NB),
        in_specs=[
            pl.BlockSpec((1, _BLK, _EMB), lambda b, i: (b, i, 0)),
            wspec, wspec, wspec, wspec, wspec,
            bspec, bspec, bspec, bspec,
        ],
        out_specs=pl.BlockSpec((1, _BLK, _EMB), lambda b, i: (b, i, 0)),
        out_shape=jax.ShapeDtypeStruct((B, _T, _EMB), jnp.float32),
        scratch_shapes=[
            pltpu.VMEM((_NM, _EMB), _BF),
            pltpu.VMEM((_NM, _EMB), _BF),
        ],
    )(x.astype(_BF), Wk.T.astype(_BF), Wq.T.astype(_BF), Wv.T.astype(_BF),
      Wu[:, :_EMB].T.astype(_BF), Wu[:, _EMB:].T.astype(_BF),
      bk.reshape(1, _EMB), bq.reshape(1, _EMB), bv.reshape(1, _EMB),
      bu.reshape(1, _EMB))


# speedup vs baseline: 270.4991x; 1.0051x over previous
import jax
import jax.numpy as jnp
from jax import lax
from jax.experimental import pallas as pl
from jax.experimental.pallas import tpu as pltpu

_EMB = 768
_T = 2048
_KK = 32
_BLK = 256
_NB = _T // _BLK
_MPB = _BLK // _KK
_NM = _T // _KK
_BF = jnp.bfloat16


def _dot(a, b):
    return jnp.dot(a, b, preferred_element_type=jnp.float32)


def _attn_kernel(x_ref, WkT_ref, WqT_ref, WvT_ref, Wu1T_ref, Wu2T_ref,
                 bk_ref, bq_ref, bv_ref, bu_ref, out_ref, qs_ref, vs_ref):
    i = pl.program_id(1)

    @pl.when(i == 0)
    def _():
        qs_ref[...] = jnp.zeros((_NM, _EMB), jnp.float32)
        vs_ref[...] = jnp.zeros((_NM, _EMB), jnp.float32)

    xb = x_ref[0]
    K = _dot(xb, WkT_ref[...]) + bk_ref[...]
    Q = _dot(xb, WqT_ref[...]) + bq_ref[...]
    V = _dot(xb, WvT_ref[...]) + bv_ref[...]
    Kb = K.astype(_BF)
    Qb = Q.astype(_BF)
    Vb = V.astype(_BF)

    qs_ref[pl.ds(i * _MPB, _MPB), :] = Q.reshape(_MPB, _KK, _EMB)[:, 0, :]
    vs_ref[pl.ds(i * _MPB, _MPB), :] = V.reshape(_MPB, _KK, _EMB)[:, 0, :]

    S = _dot(Kb, Qb.T)
    r = lax.broadcasted_iota(jnp.int32, (_BLK, _BLK), 0)
    c = lax.broadcasted_iota(jnp.int32, (_BLK, _BLK), 1)
    mask1 = (r // _KK == c // _KK) & (c <= r)
    h1 = _dot(jnp.where(mask1, S, 0.0).astype(_BF), Vb)

    D = _dot(Kb, qs_ref[...].astype(_BF).T)
    rj = lax.broadcasted_iota(jnp.int32, (_BLK, _NM), 0)
    cm = lax.broadcasted_iota(jnp.int32, (_BLK, _NM), 1)
    mask2 = (cm * _KK) <= (i * _BLK + rj)
    h2 = _dot(jnp.where(mask2, D, 0.0).astype(_BF), vs_ref[...].astype(_BF))

    out_ref[0] = (_dot(h1.astype(_BF), Wu1T_ref[...])
                  + _dot(h2.astype(_BF), Wu2T_ref[...])
                  + bu_ref[...])


def kernel(x, Wk, bk, Wq, bq, Wv, bv, Wu, bu):
    B = x.shape[0]
    wspec = pl.BlockSpec((_EMB, _EMB), lambda b, i: (0, 0))
    bspec = pl.BlockSpec((1, _EMB), lambda b, i: (0, 0))
    return pl.pallas_call(
        _attn_kernel,
        grid=(B, _NB),
        in_specs=[
            pl.BlockSpec((1, _BLK, _EMB), lambda b, i: (b, i, 0)),
            wspec, wspec, wspec, wspec, wspec,
            bspec, bspec, bspec, bspec,
        ],
        out_specs=pl.BlockSpec((1, _BLK, _EMB), lambda b, i: (b, i, 0)),
        out_shape=jax.ShapeDtypeStruct((B, _T, _EMB), jnp.float32),
        scratch_shapes=[
            pltpu.VMEM((_NM, _EMB), jnp.float32),
            pltpu.VMEM((_NM, _EMB), jnp.float32),
        ],
    )(x.astype(_BF), Wk.T.astype(_BF), Wq.T.astype(_BF), Wv.T.astype(_BF),
      Wu[:, :_EMB].T.astype(_BF), Wu[:, _EMB:].T.astype(_BF),
      bk.reshape(1, _EMB), bq.reshape(1, _EMB), bv.reshape(1, _EMB),
      bu.reshape(1, _EMB))
